# Initial kernel scaffold; baseline (speedup 1.0000x reference)
import jax
import jax.numpy as jnp
from jax.experimental import pallas as pl


def kernel(x_input, x_function, x_output, edge_index_if, edge_index_fi, edge_index_ff, edge_index_of, edge_index_fo, batch, mask, Wl0, bl0, Wr0, Wl, bl, Wr, ln_w, ln_b, att_w, lin_w, lin_b):
    raise NotImplementedError("write your pallas kernel here")


# speedup vs baseline: 7.1047x; 7.1047x over previous
import functools
import math

import jax
import jax.numpy as jnp
from jax import lax
from jax.experimental import pallas as pl
from jax.experimental.pallas import tpu as pltpu
from jax.experimental.pallas import tpu_sc as plsc

N_IN = 10000
N_FN = 50000
N_OUT = 10000
D = 128
H = 32
NH = 10
NG = 32

NCORE = 2
NSUB = 16
NW = NCORE * NSUB
SUB = 128
CHUNK = 512
KSUB = CHUNK // SUB
PADROWS = 128
ACC_F = 51200
ACC_IO = 10240
CNTW = 16
BLK = 2000

_INV_SQRT2 = 1.0 / math.sqrt(2.0)


def _gelu(x):
    return 0.5 * x * (1.0 + lax.erf(x * _INV_SQRT2))


def _round_edges(e):
    q = NW * CHUNK
    return ((e + q - 1) // q) * q


def _sc_mesh():
    return plsc.VectorSubcoreMesh(core_axis_name="c", subcore_axis_name="s")


def _zero_acc(acc, zv, si, nrows):
    zper = nrows // NSUB // SUB

    def zb(i, c):
        pltpu.sync_copy(zv, acc.at[pl.ds((si * zper + i) * SUB, SUB)])
        return c

    lax.fori_loop(0, zper, zb, 0)


def _flush_acc(acc, out, ci, si, acc_rows):
    share = acc_rows // NSUB
    pltpu.sync_copy(acc.at[pl.ds(si * share, share)],
                    out.at[pl.ds(ci * acc_rows + si * share, share)])


def _agg_rel(P, s2d, d2d, out, n_dst, acc_rows, ci, si,
             acc, sidx, didx, rows, zv, gsem):
    _zero_acc(acc, zv, si, acc_rows)
    plsc.subcore_barrier()

    ew = (s2d.shape[0] * SUB) // NW
    nchunks = ew // CHUNK
    wrow = (ci * NSUB + si) * (ew // SUB)

    def cb(k, c):
        r0 = wrow + k * KSUB
        pltpu.sync_copy(s2d.at[pl.ds(r0, KSUB)], sidx)
        pltpu.sync_copy(d2d.at[pl.ds(r0, KSUB)], didx)
        cps = [pltpu.async_copy(P.at[sidx.at[j]],
                                rows.at[pl.ds(j * SUB, SUB)], gsem)
               for j in range(KSUB)]
        for cp in cps:
            cp.wait()
        for j in range(KSUB):
            pltpu.sync_copy(rows.at[pl.ds(j * SUB, SUB)],
                            acc.at[didx.at[j]], add=True)
        return c

    lax.fori_loop(0, nchunks, cb, 0)
    plsc.subcore_barrier()
    _flush_acc(acc, out, ci, si, acc_rows)
    plsc.subcore_barrier()


def _build_sc_layer(shapes_sd, last):
    rels = ["if", "ff", "of"] + ([] if last else ["fi", "fo"])
    ndst = {"if": N_FN, "ff": N_FN, "of": N_FN, "fi": N_IN, "fo": N_OUT}
    accr = {"if": ACC_F, "ff": ACC_F, "of": ACC_F, "fi": ACC_IO, "fo": ACC_IO}

    out_type = [jax.ShapeDtypeStruct((NCORE * accr[r], H), jnp.float32)
                for r in rels]

    @functools.partial(
        pl.kernel,
        out_type=out_type,
        mesh=_sc_mesh(),
        compiler_params=pltpu.CompilerParams(use_tc_tiling_on_sc=False),
        scratch_types=[
            pltpu.VMEM_SHARED((ACC_F, H), jnp.float32),
            pltpu.VMEM((KSUB, SUB), jnp.int32),
            pltpu.VMEM((KSUB, SUB), jnp.int32),
            pltpu.VMEM((CHUNK, H), jnp.float32),
            pltpu.VMEM((SUB, H), jnp.float32),
            pltpu.SemaphoreType.DMA,
        ],
    )
    def k(*args):
        nr = len(rels)
        Ps = args[0:nr]
        s2ds = args[nr:2 * nr]
        d2ds = args[2 * nr:3 * nr]
        zeros_hbm = args[3 * nr]
        outs = args[3 * nr + 1:3 * nr + 1 + nr]
        acc, sidx, didx, rows, zv, gsem = args[3 * nr + 1 + nr:]
        ci = lax.axis_index("c")
        si = lax.axis_index("s")
        pltpu.sync_copy(zeros_hbm, zv)
        for t, r in enumerate(rels):
            _agg_rel(Ps[t], s2ds[t], d2ds[t], outs[t], ndst[r], accr[r],
                     ci, si, acc, sidx, didx, rows, zv, gsem)

    return k, rels


def _build_sc_counts():
    rels = ["if", "ff", "of", "fi", "fo"]
    ndst = {"if": N_FN, "ff": N_FN, "of": N_FN, "fi": N_IN, "fo": N_OUT}
    accr = {"if": ACC_F, "ff": ACC_F, "of": ACC_F, "fi": ACC_IO, "fo": ACC_IO}

    out_type = [jax.ShapeDtypeStruct((NCORE * accr[r], CNTW), jnp.float32)
                for r in rels]

    @functools.partial(
        pl.kernel,
        out_type=out_type,
        mesh=_sc_mesh(),
        compiler_params=pltpu.CompilerParams(use_tc_tiling_on_sc=False),
        scratch_types=[
            pltpu.VMEM_SHARED((ACC_F, CNTW), jnp.float32),
            pltpu.VMEM((KSUB, SUB), jnp.int32),
            pltpu.VMEM((SUB, CNTW), jnp.float32),
            pltpu.VMEM((SUB, CNTW), jnp.float32),
        ],
    )
    def k(*args):
        d2ds = args[0:5]
        zeros_hbm = args[5]
        ones_hbm = args[6]
        outs = args[7:12]
        acc, didx, zv, ones_v = args[12:]
        ci = lax.axis_index("c")
        si = lax.axis_index("s")
        pltpu.sync_copy(zeros_hbm, zv)
        pltpu.sync_copy(ones_hbm, ones_v)
        for t, r in enumerate(rels):
            _zero_acc(acc, zv, si, accr[r])
            plsc.subcore_barrier()
            d2d = d2ds[t]
            ew = (d2d.shape[0] * SUB) // NW
            nchunks = ew // CHUNK
            wrow = (ci * NSUB + si) * (ew // SUB)

            def cb(kk, c, d2d=d2d, wrow=wrow):
                r0 = wrow + kk * KSUB
                pltpu.sync_copy(d2d.at[pl.ds(r0, KSUB)], didx)
                for j in range(KSUB):
                    pltpu.sync_copy(ones_v, acc.at[didx.at[j]], add=True)
                return c

            lax.fori_loop(0, nchunks, cb, 0)
            plsc.subcore_barrier()
            _flush_acc(acc, outs[t], ci, si, accr[r])
            plsc.subcore_barrier()

    return k


def _dot(a, b):
    return jnp.dot(a, b, preferred_element_type=jnp.float32)


def _prep0(x, mask2, w, nout):
    n = x.shape[0]
    grid = n // BLK
    have_mask = mask2 is not None

    def body(*refs):
        if have_mask:
            x_ref, m_ref, w_ref = refs[:3]
            outs = refs[3:]
            xv = x_ref[...] * m_ref[...]
        else:
            x_ref, w_ref = refs[:2]
            outs = refs[2:]
            xv = x_ref[...]
        y = _dot(xv, w_ref[...])
        for t in range(nout):
            outs[t][...] = y[:, H * t:H * (t + 1)]

    in_specs = [pl.BlockSpec((BLK, D), lambda i: (i, 0))]
    args = [x]
    if have_mask:
        in_specs.append(pl.BlockSpec((BLK, 1), lambda i: (i, 0)))
        args.append(mask2)
    in_specs.append(pl.BlockSpec((D, H * nout), lambda i: (0, 0)))
    args.append(w)

    return pl.pallas_call(
        body,
        grid=(grid,),
        in_specs=in_specs,
        out_specs=[pl.BlockSpec((BLK, H), lambda i: (i, 0))] * nout,
        out_shape=[jax.ShapeDtypeStruct((n, H), jnp.float32)] * nout,
    )(*args)


def _mean_blk(a_ref, c_ref):
    s = a_ref[0] + a_ref[1]
    cnt = c_ref[0][:, 0:1] + c_ref[1][:, 0:1]
    return s / jnp.maximum(cnt, 1.0)


def _ln_gelu(nf, lw_ref, lb_ref):
    g = _gelu(nf)
    mu = jnp.mean(g, axis=-1, keepdims=True)
    var = jnp.mean((g - mu) ** 2, axis=-1, keepdims=True)
    return (g - mu) / jnp.sqrt(var + 1e-5) * lw_ref[...] + lb_ref[...]


def _combine_f(aggs, cnts, r, mask2, bsum, lnw, lnb, w, nout):
    n = r.shape[0]
    grid = n // BLK

    def body(a0, c0, a1, c1, a2, c2, r_ref, m_ref, b_ref, lw, lb, *rest):
        if w is not None:
            w_ref = rest[0]
            outs = rest[1:]
        else:
            outs = rest
        nf = (_mean_blk(a0, c0) + _mean_blk(a1, c1) + _mean_blk(a2, c2)
              + r_ref[...] + b_ref[...])
        nf = nf * m_ref[...]
        x = _ln_gelu(nf, lw, lb)
        if w is not None:
            y = _dot(x, w_ref[...])
            for t in range(nout):
                outs[t][...] = y[:, H * t:H * (t + 1)]
        else:
            outs[0][...] = x

    a_spec = pl.BlockSpec((NCORE, BLK, H), lambda i: (0, i, 0))
    c_spec = pl.BlockSpec((NCORE, BLK, CNTW), lambda i: (0, i, 0))
    v_spec = pl.BlockSpec((BLK, H), lambda i: (i, 0))
    s_spec = pl.BlockSpec((1, H), lambda i: (0, 0))
    in_specs = [a_spec, c_spec, a_spec, c_spec, a_spec, c_spec,
                v_spec, pl.BlockSpec((BLK, 1), lambda i: (i, 0)),
                s_spec, s_spec, s_spec]
    args = [aggs[0], cnts[0], aggs[1], cnts[1], aggs[2], cnts[2],
            r, mask2, bsum, lnw, lnb]
    if w is not None:
        in_specs.append(pl.BlockSpec((H, H * nout), lambda i: (0, 0)))
        args.append(w)

    return pl.pallas_call(
        body,
        grid=(grid,),
        in_specs=in_specs,
        out_specs=[v_spec] * nout,
        out_shape=[jax.ShapeDtypeStruct((n, H), jnp.float32)] * nout,
    )(*args)


def _combine_io(agg, cnt, r, bsum, lnw, lnb, w, nout):
    n = r.shape[0]
    grid = n // BLK

    def body(a0, c0, r_ref, b_ref, lw, lb, w_ref, *outs):
        nf = _mean_blk(a0, c0) + r_ref[...] + b_ref[...]
        x = _ln_gelu(nf, lw, lb)
        y = _dot(x, w_ref[...])
        for t in range(nout):
            outs[t][...] = y[:, H * t:H * (t + 1)]

    v_spec = pl.BlockSpec((BLK, H), lambda i: (i, 0))
    s_spec = pl.BlockSpec((1, H), lambda i: (0, 0))
    in_specs = [pl.BlockSpec((NCORE, BLK, H), lambda i: (0, i, 0)),
                pl.BlockSpec((NCORE, BLK, CNTW), lambda i: (0, i, 0)),
                v_spec, s_spec, s_spec, s_spec,
                pl.BlockSpec((H, H * nout), lambda i: (0, 0))]

    return pl.pallas_call(
        body,
        grid=(grid,),
        in_specs=in_specs,
        out_specs=[v_spec] * nout,
        out_shape=[jax.ShapeDtypeStruct((n, H), jnp.float32)] * nout,
    )(agg, cnt, r, bsum, lnw, lnb, w)


def _pool(xf, mask2, batch2, att_w, lin_w, lin_b):
    nblk = N_FN // BLK

    def body(xf_ref, m_ref, b_ref, aw_ref, lw_ref, lb_ref, o_ref,
             smax_s, den_s, num_s):
        p = pl.program_id(0)
        i = pl.program_id(1)
        xfm = xf_ref[...] * m_ref[...]
        gid = lax.broadcasted_iota(jnp.int32, (BLK, NG), 1)
        oneh = (b_ref[...] == gid).astype(jnp.float32)
        scores = _dot(xfm, aw_ref[...])
        neg = jnp.float32(-jnp.inf)

        @pl.when(p == 0)
        def _phase0():
            rows = []
            for g in range(NG):
                mg = jnp.where(oneh[:, g:g + 1] > 0.0, scores, neg)
                rows.append(jnp.max(mg, axis=0, keepdims=True))
            bm = jnp.concatenate(rows, axis=0)

            @pl.when(i == 0)
            def _():
                smax_s[...] = bm

            @pl.when(i > 0)
            def _():
                smax_s[...] = jnp.maximum(smax_s[...], bm)

        @pl.when(p == 1)
        def _phase1():
            smax = smax_s[...]
            smax = jnp.where(jnp.isfinite(smax), smax, 0.0)
            shift = _dot(oneh, smax)
            ex = jnp.exp(scores - shift)
            den = lax.dot_general(oneh, ex, (((0,), (0,)), ((), ())),
                                  preferred_element_type=jnp.float32)
            nums = []
            for h in range(NH):
                wh = oneh * ex[:, h:h + 1]
                nums.append(lax.dot_general(wh, xfm, (((0,), (0,)), ((), ())),
                                            preferred_element_type=jnp.float32))
            num = jnp.concatenate(nums, axis=0)

            @pl.when(i == 0)
            def _():
                den_s[...] = den
                num_s[...] = num

            @pl.when(i > 0)
            def _():
                den_s[...] = den_s[...] + den
                num_s[...] = num_s[...] + num

        @pl.when((p == 1) & (i == nblk - 1))
        def _epilogue():
            den = jnp.maximum(den_s[...], 1e-9)
            acc = jnp.zeros((NG, 1), jnp.float32)
            for h in range(NH):
                ph = num_s[h * NG:(h + 1) * NG, :] / den[:, h:h + 1]
                acc = acc + _dot(_gelu(ph), lw_ref[h * H:(h + 1) * H, :])
            o_ref[...] = acc + lb_ref[...]

    return pl.pallas_call(
        body,
        grid=(2, nblk),
        in_specs=[pl.BlockSpec((BLK, H), lambda p, i: (i, 0)),
                  pl.BlockSpec((BLK, 1), lambda p, i: (i, 0)),
                  pl.BlockSpec((BLK, 1), lambda p, i: (i, 0)),
                  pl.BlockSpec(att_w.shape, lambda p, i: (0, 0)),
                  pl.BlockSpec(lin_w.shape, lambda p, i: (0, 0)),
                  pl.BlockSpec(lin_b.shape, lambda p, i: (0, 0))],
        out_specs=pl.BlockSpec((NG, 1), lambda p, i: (0, 0)),
        out_shape=jax.ShapeDtypeStruct((NG, 1), jnp.float32),
        scratch_shapes=[pltpu.VMEM((NG, NH), jnp.float32),
                        pltpu.VMEM((NG, NH), jnp.float32),
                        pltpu.VMEM((NH * NG, H), jnp.float32)],
    )(xf, mask2, batch2, att_w, lin_w, lin_b)


def kernel(x_input, x_function, x_output, edge_index_if, edge_index_fi,
           edge_index_ff, edge_index_of, edge_index_fo, batch, mask,
           Wl0, bl0, Wr0, Wl, bl, Wr, ln_w, ln_b, att_w, lin_w, lin_b):
    f32 = jnp.float32
    mask2 = mask[:, None].astype(f32)
    batch2 = batch[:, None].astype(jnp.int32)
    zeros32 = jnp.zeros((SUB, H), f32)
    zeros16 = jnp.zeros((SUB, CNTW), f32)
    ones16 = jnp.ones((SUB, CNTW), f32)

    def prep_edges(ei, n_src, n_dst):
        e = ei.shape[1]
        ep = _round_edges(e)
        pad = ep - e
        ar = jnp.arange(pad, dtype=jnp.int32)
        s = jnp.concatenate([ei[0].astype(jnp.int32), ar % n_src])
        dd = jnp.concatenate([ei[1].astype(jnp.int32),
                              n_dst + (ar % PADROWS)])
        return s.reshape(ep // SUB, SUB), dd.reshape(ep // SUB, SUB)

    sif, dif = prep_edges(edge_index_if, N_IN, N_FN)
    sff, dff = prep_edges(edge_index_ff, N_FN, N_FN)
    sof, dof = prep_edges(edge_index_of, N_OUT, N_FN)
    sfi, dfi = prep_edges(edge_index_fi, N_FN, N_IN)
    sfo, dfo = prep_edges(edge_index_fo, N_FN, N_OUT)

    counts_k = _build_sc_counts()
    cr = counts_k(dif, dff, dof, dfi, dfo, zeros16, ones16)
    c_if, c_ff, c_of, c_fi, c_fo = [
        c.reshape(NCORE, n, CNTW)
        for c, n in zip(cr, (ACC_F, ACC_F, ACC_F, ACC_IO, ACC_IO))]

    def wf_cat(WL, WR):
        return jnp.concatenate([WL[2], WL[1], WL[4], WR[0] + WR[2] + WR[3]],
                               axis=1)

    def wi_cat(WL, WR):
        return jnp.concatenate([WL[0], WR[1]], axis=1)

    def wo_cat(WL, WR):
        return jnp.concatenate([WL[3], WR[4]], axis=1)

    Wf = [wf_cat(Wl0, Wr0)] + [wf_cat(Wl[t], Wr[t]) for t in range(4)]
    Wi = [wi_cat(Wl0, Wr0)] + [wi_cat(Wl[t], Wr[t]) for t in range(4)]
    Wo = [wo_cat(Wl0, Wr0)] + [wo_cat(Wl[t], Wr[t]) for t in range(4)]
    bsf = [(bl0[0] + bl0[2] + bl0[3])[None, :]] + \
          [(bl[t, 0] + bl[t, 2] + bl[t, 3])[None, :] for t in range(4)]
    bsi = [bl0[1][None, :]] + [bl[t, 1][None, :] for t in range(4)]
    bso = [bl0[4][None, :]] + [bl[t, 4][None, :] for t in range(4)]
    lnw2 = ln_w[None, :]
    lnb2 = ln_b[None, :]

    Pff, Pfi, Pfo, Rf = _prep0(x_function, mask2, Wf[0], 4)
    Pif, Ri = _prep0(x_input, None, Wi[0], 2)
    Pof, Ro = _prep0(x_output, None, Wo[0], 2)

    sd_full = None
    layer_full, rels_full = _build_sc_layer(sd_full, last=False)
    layer_last, rels_last = _build_sc_layer(sd_full, last=True)

    xf_fin = None
    for l in range(5):
        last = l == 4
        if not last:
            o_if, o_ff, o_of, o_fi, o_fo = layer_full(
                Pif, Pff, Pof, Pfi, Pfo,
                sif, sff, sof, sfi, sfo,
                dif, dff, dof, dfi, dfo, zeros32)
            a_if = o_if.reshape(NCORE, ACC_F, H)
            a_ff = o_ff.reshape(NCORE, ACC_F, H)
            a_of = o_of.reshape(NCORE, ACC_F, H)
            a_fi = o_fi.reshape(NCORE, ACC_IO, H)
            a_fo = o_fo.reshape(NCORE, ACC_IO, H)
            if l < 3:
                Pff, Pfi, Pfo, Rf = _combine_f(
                    (a_if, a_ff, a_of), (c_if, c_ff, c_of), Rf, mask2,
                    bsf[l], lnw2, lnb2, Wf[l + 1], 4)
                Pif, Ri = _combine_io(a_fi, c_fi, Ri, bsi[l], lnw2, lnb2,
                                      Wi[l + 1], 2)
                Pof, Ro = _combine_io(a_fo, c_fo, Ro, bso[l], lnw2, lnb2,
                                      Wo[l + 1], 2)
            else:
                w4 = jnp.concatenate([Wl[3][2], Wr[3][0] + Wr[3][2] + Wr[3][3]],
                                     axis=1)
                Pff, Rf = _combine_f(
                    (a_if, a_ff, a_of), (c_if, c_ff, c_of), Rf, mask2,
                    bsf[3], lnw2, lnb2, w4, 2)
                (Pif,) = _combine_io(a_fi, c_fi, Ri, bsi[3], lnw2, lnb2,
                                     Wl[3][0], 1)
                (Pof,) = _combine_io(a_fo, c_fo, Ro, bso[3], lnw2, lnb2,
                                     Wl[3][3], 1)
        else:
            o_if, o_ff, o_of = layer_last(
                Pif, Pff, Pof, sif, sff, sof, dif, dff, dof, zeros32)
            a_if = o_if.reshape(NCORE, ACC_F, H)
            a_ff = o_ff.reshape(NCORE, ACC_F, H)
            a_of = o_of.reshape(NCORE, ACC_F, H)
            (xf_fin,) = _combine_f(
                (a_if, a_ff, a_of), (c_if, c_ff, c_of), Rf, mask2,
                bsf[4], lnw2, lnb2, None, 1)

    return _pool(xf_fin, mask2, batch2, att_w,
                 lin_w, lin_b[None, :].astype(f32))


# speedup vs baseline: 7.2100x; 1.0148x over previous
import functools
import math

import jax
import jax.numpy as jnp
from jax import lax
from jax.experimental import pallas as pl
from jax.experimental.pallas import tpu as pltpu
from jax.experimental.pallas import tpu_sc as plsc

N_IN = 10000
N_FN = 50000
N_OUT = 10000
D = 128
H = 32
NH = 10
NG = 32

NCORE = 2
NSUB = 16
NW = NCORE * NSUB
SUB = 128
CHUNK = 256
KSUB = CHUNK // SUB
PADROWS = 128
ACC_F = 51200
ACC_IO = 10240
CNTW = 16
BLK = 2000

_INV_SQRT2 = 1.0 / math.sqrt(2.0)


def _gelu(x):
    return 0.5 * x * (1.0 + lax.erf(x * _INV_SQRT2))


def _round_edges(e):
    q = 2 * NW * CHUNK
    return ((e + q - 1) // q) * q


def _sc_mesh():
    return plsc.VectorSubcoreMesh(core_axis_name="c", subcore_axis_name="s")


def _zero_acc(acc, zv, si, nrows):
    zper = nrows // NSUB // SUB

    def zb(i, c):
        pltpu.sync_copy(zv, acc.at[pl.ds((si * zper + i) * SUB, SUB)])
        return c

    lax.fori_loop(0, zper, zb, 0)


def _flush_acc(acc, out, ci, si, acc_rows):
    share = acc_rows // NSUB
    pltpu.sync_copy(acc.at[pl.ds(si * share, share)],
                    out.at[pl.ds(ci * acc_rows + si * share, share)])


def _agg_rel(P, s2d, d2d, out, n_dst, acc_rows, ci, si,
             acc, sidx, didx, rows, zv, sem0, sem1):
    _zero_acc(acc, zv, si, acc_rows)
    plsc.subcore_barrier()

    ew = (s2d.shape[0] * SUB) // NW
    nchunks = ew // CHUNK
    wrow = (ci * NSUB + si) * (ew // SUB)
    sems = (sem0, sem1)

    def fire(k, p):
        r0 = wrow + k * KSUB
        pltpu.sync_copy(s2d.at[pl.ds(r0, KSUB)],
                        sidx.at[pl.ds(p * KSUB, KSUB)])
        pltpu.sync_copy(d2d.at[pl.ds(r0, KSUB)],
                        didx.at[pl.ds(p * KSUB, KSUB)])
        for j in range(KSUB):
            pltpu.async_copy(P.at[sidx.at[p * KSUB + j]],
                             rows.at[pl.ds((p * KSUB + j) * SUB, SUB)],
                             sems[p])

    def drain_scatter(p):
        for j in range(KSUB):
            pltpu.make_async_copy(
                P.at[sidx.at[p * KSUB + j]],
                rows.at[pl.ds((p * KSUB + j) * SUB, SUB)], sems[p]).wait()
            pltpu.sync_copy(rows.at[pl.ds((p * KSUB + j) * SUB, SUB)],
                            acc.at[didx.at[p * KSUB + j]], add=True)

    def cb(i, c):
        fire(2 * i, 0)

        @pl.when(i > 0)
        def _():
            drain_scatter(1)
        fire(2 * i + 1, 1)
        drain_scatter(0)
        return c

    lax.fori_loop(0, nchunks // 2, cb, 0)
    drain_scatter(1)
    plsc.subcore_barrier()
    _flush_acc(acc, out, ci, si, acc_rows)
    plsc.subcore_barrier()


def _build_sc_layer(shapes_sd, last):
    rels = ["if", "ff", "of"] + ([] if last else ["fi", "fo"])
    ndst = {"if": N_FN, "ff": N_FN, "of": N_FN, "fi": N_IN, "fo": N_OUT}
    accr = {"if": ACC_F, "ff": ACC_F, "of": ACC_F, "fi": ACC_IO, "fo": ACC_IO}

    out_type = [jax.ShapeDtypeStruct((NCORE * accr[r], H), jnp.float32)
                for r in rels]

    @functools.partial(
        pl.kernel,
        out_type=out_type,
        mesh=_sc_mesh(),
        compiler_params=pltpu.CompilerParams(use_tc_tiling_on_sc=False),
        scratch_types=[
            pltpu.VMEM_SHARED((ACC_F, H), jnp.float32),
            pltpu.VMEM((2 * KSUB, SUB), jnp.int32),
            pltpu.VMEM((2 * KSUB, SUB), jnp.int32),
            pltpu.VMEM((2 * CHUNK, H), jnp.float32),
            pltpu.VMEM((SUB, H), jnp.float32),
            pltpu.SemaphoreType.DMA,
            pltpu.SemaphoreType.DMA,
        ],
    )
    def k(*args):
        nr = len(rels)
        Ps = args[0:nr]
        s2ds = args[nr:2 * nr]
        d2ds = args[2 * nr:3 * nr]
        zeros_hbm = args[3 * nr]
        outs = args[3 * nr + 1:3 * nr + 1 + nr]
        acc, sidx, didx, rows, zv, sem0, sem1 = args[3 * nr + 1 + nr:]
        ci = lax.axis_index("c")
        si = lax.axis_index("s")
        pltpu.sync_copy(zeros_hbm, zv)
        for t, r in enumerate(rels):
            _agg_rel(Ps[t], s2ds[t], d2ds[t], outs[t], ndst[r], accr[r],
                     ci, si, acc, sidx, didx, rows, zv, sem0, sem1)

    return k, rels


def _build_sc_counts():
    rels = ["if", "ff", "of", "fi", "fo"]
    ndst = {"if": N_FN, "ff": N_FN, "of": N_FN, "fi": N_IN, "fo": N_OUT}
    accr = {"if": ACC_F, "ff": ACC_F, "of": ACC_F, "fi": ACC_IO, "fo": ACC_IO}

    out_type = [jax.ShapeDtypeStruct((NCORE * accr[r], CNTW), jnp.float32)
                for r in rels]

    @functools.partial(
        pl.kernel,
        out_type=out_type,
        mesh=_sc_mesh(),
        compiler_params=pltpu.CompilerParams(use_tc_tiling_on_sc=False),
        scratch_types=[
            pltpu.VMEM_SHARED((ACC_F, CNTW), jnp.float32),
            pltpu.VMEM((KSUB, SUB), jnp.int32),
            pltpu.VMEM((SUB, CNTW), jnp.float32),
            pltpu.VMEM((SUB, CNTW), jnp.float32),
        ],
    )
    def k(*args):
        d2ds = args[0:5]
        zeros_hbm = args[5]
        ones_hbm = args[6]
        outs = args[7:12]
        acc, didx, zv, ones_v = args[12:]
        ci = lax.axis_index("c")
        si = lax.axis_index("s")
        pltpu.sync_copy(zeros_hbm, zv)
        pltpu.sync_copy(ones_hbm, ones_v)
        for t, r in enumerate(rels):
            _zero_acc(acc, zv, si, accr[r])
            plsc.subcore_barrier()
            d2d = d2ds[t]
            ew = (d2d.shape[0] * SUB) // NW
            nchunks = ew // CHUNK
            wrow = (ci * NSUB + si) * (ew // SUB)

            def cb(kk, c, d2d=d2d, wrow=wrow):
                r0 = wrow + kk * KSUB
                pltpu.sync_copy(d2d.at[pl.ds(r0, KSUB)], didx)
                for j in range(KSUB):
                    pltpu.sync_copy(ones_v, acc.at[didx.at[j]], add=True)
                return c

            lax.fori_loop(0, nchunks, cb, 0)
            plsc.subcore_barrier()
            _flush_acc(acc, outs[t], ci, si, accr[r])
            plsc.subcore_barrier()

    return k


def _dot(a, b):
    return jnp.dot(a, b, preferred_element_type=jnp.float32)


def _prep0(x, mask2, w, nout):
    n = x.shape[0]
    grid = n // BLK
    have_mask = mask2 is not None

    def body(*refs):
        if have_mask:
            x_ref, m_ref, w_ref = refs[:3]
            outs = refs[3:]
            xv = x_ref[...] * m_ref[...]
        else:
            x_ref, w_ref = refs[:2]
            outs = refs[2:]
            xv = x_ref[...]
        y = _dot(xv, w_ref[...])
        for t in range(nout):
            outs[t][...] = y[:, H * t:H * (t + 1)]

    in_specs = [pl.BlockSpec((BLK, D), lambda i: (i, 0))]
    args = [x]
    if have_mask:
        in_specs.append(pl.BlockSpec((BLK, 1), lambda i: (i, 0)))
        args.append(mask2)
    in_specs.append(pl.BlockSpec((D, H * nout), lambda i: (0, 0)))
    args.append(w)

    return pl.pallas_call(
        body,
        grid=(grid,),
        in_specs=in_specs,
        out_specs=[pl.BlockSpec((BLK, H), lambda i: (i, 0))] * nout,
        out_shape=[jax.ShapeDtypeStruct((n, H), jnp.float32)] * nout,
    )(*args)


def _mean_blk(a_ref, c_ref):
    s = a_ref[0] + a_ref[1]
    cnt = c_ref[0][:, 0:1] + c_ref[1][:, 0:1]
    return s / jnp.maximum(cnt, 1.0)


def _ln_gelu(nf, lw_ref, lb_ref):
    g = _gelu(nf)
    mu = jnp.mean(g, axis=-1, keepdims=True)
    var = jnp.mean((g - mu) ** 2, axis=-1, keepdims=True)
    return (g - mu) / jnp.sqrt(var + 1e-5) * lw_ref[...] + lb_ref[...]


def _combine_f(aggs, cnts, r, mask2, bsum, lnw, lnb, w, nout):
    n = r.shape[0]
    grid = n // BLK

    def body(a0, c0, a1, c1, a2, c2, r_ref, m_ref, b_ref, lw, lb, *rest):
        if w is not None:
            w_ref = rest[0]
            outs = rest[1:]
        else:
            outs = rest
        nf = (_mean_blk(a0, c0) + _mean_blk(a1, c1) + _mean_blk(a2, c2)
              + r_ref[...] + b_ref[...])
        nf = nf * m_ref[...]
        x = _ln_gelu(nf, lw, lb)
        if w is not None:
            y = _dot(x, w_ref[...])
            for t in range(nout):
                outs[t][...] = y[:, H * t:H * (t + 1)]
        else:
            outs[0][...] = x

    a_spec = pl.BlockSpec((NCORE, BLK, H), lambda i: (0, i, 0))
    c_spec = pl.BlockSpec((NCORE, BLK, CNTW), lambda i: (0, i, 0))
    v_spec = pl.BlockSpec((BLK, H), lambda i: (i, 0))
    s_spec = pl.BlockSpec((1, H), lambda i: (0, 0))
    in_specs = [a_spec, c_spec, a_spec, c_spec, a_spec, c_spec,
                v_spec, pl.BlockSpec((BLK, 1), lambda i: (i, 0)),
                s_spec, s_spec, s_spec]
    args = [aggs[0], cnts[0], aggs[1], cnts[1], aggs[2], cnts[2],
            r, mask2, bsum, lnw, lnb]
    if w is not None:
        in_specs.append(pl.BlockSpec((H, H * nout), lambda i: (0, 0)))
        args.append(w)

    return pl.pallas_call(
        body,
        grid=(grid,),
        in_specs=in_specs,
        out_specs=[v_spec] * nout,
        out_shape=[jax.ShapeDtypeStruct((n, H), jnp.float32)] * nout,
    )(*args)


def _combine_io(agg, cnt, r, bsum, lnw, lnb, w, nout):
    n = r.shape[0]
    grid = n // BLK

    def body(a0, c0, r_ref, b_ref, lw, lb, w_ref, *outs):
        nf = _mean_blk(a0, c0) + r_ref[...] + b_ref[...]
        x = _ln_gelu(nf, lw, lb)
        y = _dot(x, w_ref[...])
        for t in range(nout):
            outs[t][...] = y[:, H * t:H * (t + 1)]

    v_spec = pl.BlockSpec((BLK, H), lambda i: (i, 0))
    s_spec = pl.BlockSpec((1, H), lambda i: (0, 0))
    in_specs = [pl.BlockSpec((NCORE, BLK, H), lambda i: (0, i, 0)),
                pl.BlockSpec((NCORE, BLK, CNTW), lambda i: (0, i, 0)),
                v_spec, s_spec, s_spec, s_spec,
                pl.BlockSpec((H, H * nout), lambda i: (0, 0))]

    return pl.pallas_call(
        body,
        grid=(grid,),
        in_specs=in_specs,
        out_specs=[v_spec] * nout,
        out_shape=[jax.ShapeDtypeStruct((n, H), jnp.float32)] * nout,
    )(agg, cnt, r, bsum, lnw, lnb, w)


def _pool(xf, mask2, batch2, att_w, lin_w, lin_b):
    nblk = N_FN // BLK

    def body(xf_ref, m_ref, b_ref, aw_ref, lw_ref, lb_ref, o_ref,
             smax_s, den_s, num_s):
        p = pl.program_id(0)
        i = pl.program_id(1)
        xfm = xf_ref[...] * m_ref[...]
        gid = lax.broadcasted_iota(jnp.int32, (BLK, NG), 1)
        oneh = (b_ref[...] == gid).astype(jnp.float32)
        scores = _dot(xfm, aw_ref[...])
        neg = jnp.float32(-jnp.inf)

        @pl.when(p == 0)
        def _phase0():
            rows = []
            for g in range(NG):
                mg = jnp.where(oneh[:, g:g + 1] > 0.0, scores, neg)
                rows.append(jnp.max(mg, axis=0, keepdims=True))
            bm = jnp.concatenate(rows, axis=0)

            @pl.when(i == 0)
            def _():
                smax_s[...] = bm

            @pl.when(i > 0)
            def _():
                smax_s[...] = jnp.maximum(smax_s[...], bm)

        @pl.when(p == 1)
        def _phase1():
            smax = smax_s[...]
            smax = jnp.where(jnp.isfinite(smax), smax, 0.0)
            shift = _dot(oneh, smax)
            ex = jnp.exp(scores - shift)
            den = lax.dot_general(oneh, ex, (((0,), (0,)), ((), ())),
                                  preferred_element_type=jnp.float32)
            nums = []
            for h in range(NH):
                wh = oneh * ex[:, h:h + 1]
                nums.append(lax.dot_general(wh, xfm, (((0,), (0,)), ((), ())),
                                            preferred_element_type=jnp.float32))
            num = jnp.concatenate(nums, axis=0)

            @pl.when(i == 0)
            def _():
                den_s[...] = den
                num_s[...] = num

            @pl.when(i > 0)
            def _():
                den_s[...] = den_s[...] + den
                num_s[...] = num_s[...] + num

        @pl.when((p == 1) & (i == nblk - 1))
        def _epilogue():
            den = jnp.maximum(den_s[...], 1e-9)
            acc = jnp.zeros((NG, 1), jnp.float32)
            for h in range(NH):
                ph = num_s[h * NG:(h + 1) * NG, :] / den[:, h:h + 1]
                acc = acc + _dot(_gelu(ph), lw_ref[h * H:(h + 1) * H, :])
            o_ref[...] = acc + lb_ref[...]

    return pl.pallas_call(
        body,
        grid=(2, nblk),
        in_specs=[pl.BlockSpec((BLK, H), lambda p, i: (i, 0)),
                  pl.BlockSpec((BLK, 1), lambda p, i: (i, 0)),
                  pl.BlockSpec((BLK, 1), lambda p, i: (i, 0)),
                  pl.BlockSpec(att_w.shape, lambda p, i: (0, 0)),
                  pl.BlockSpec(lin_w.shape, lambda p, i: (0, 0)),
                  pl.BlockSpec(lin_b.shape, lambda p, i: (0, 0))],
        out_specs=pl.BlockSpec((NG, 1), lambda p, i: (0, 0)),
        out_shape=jax.ShapeDtypeStruct((NG, 1), jnp.float32),
        scratch_shapes=[pltpu.VMEM((NG, NH), jnp.float32),
                        pltpu.VMEM((NG, NH), jnp.float32),
                        pltpu.VMEM((NH * NG, H), jnp.float32)],
    )(xf, mask2, batch2, att_w, lin_w, lin_b)


def kernel(x_input, x_function, x_output, edge_index_if, edge_index_fi,
           edge_index_ff, edge_index_of, edge_index_fo, batch, mask,
           Wl0, bl0, Wr0, Wl, bl, Wr, ln_w, ln_b, att_w, lin_w, lin_b):
    f32 = jnp.float32
    mask2 = mask[:, None].astype(f32)
    batch2 = batch[:, None].astype(jnp.int32)
    zeros32 = jnp.zeros((SUB, H), f32)
    zeros16 = jnp.zeros((SUB, CNTW), f32)
    ones16 = jnp.ones((SUB, CNTW), f32)

    def prep_edges(ei, n_src, n_dst):
        e = ei.shape[1]
        ep = _round_edges(e)
        pad = ep - e
        ar = jnp.arange(pad, dtype=jnp.int32)
        s = jnp.concatenate([ei[0].astype(jnp.int32), ar % n_src])
        dd = jnp.concatenate([ei[1].astype(jnp.int32),
                              n_dst + (ar % PADROWS)])
        return s.reshape(ep // SUB, SUB), dd.reshape(ep // SUB, SUB)

    sif, dif = prep_edges(edge_index_if, N_IN, N_FN)
    sff, dff = prep_edges(edge_index_ff, N_FN, N_FN)
    sof, dof = prep_edges(edge_index_of, N_OUT, N_FN)
    sfi, dfi = prep_edges(edge_index_fi, N_FN, N_IN)
    sfo, dfo = prep_edges(edge_index_fo, N_FN, N_OUT)

    counts_k = _build_sc_counts()
    cr = counts_k(dif, dff, dof, dfi, dfo, zeros16, ones16)
    c_if, c_ff, c_of, c_fi, c_fo = [
        c.reshape(NCORE, n, CNTW)
        for c, n in zip(cr, (ACC_F, ACC_F, ACC_F, ACC_IO, ACC_IO))]

    def wf_cat(WL, WR):
        return jnp.concatenate([WL[2], WL[1], WL[4], WR[0] + WR[2] + WR[3]],
                               axis=1)

    def wi_cat(WL, WR):
        return jnp.concatenate([WL[0], WR[1]], axis=1)

    def wo_cat(WL, WR):
        return jnp.concatenate([WL[3], WR[4]], axis=1)

    Wf = [wf_cat(Wl0, Wr0)] + [wf_cat(Wl[t], Wr[t]) for t in range(4)]
    Wi = [wi_cat(Wl0, Wr0)] + [wi_cat(Wl[t], Wr[t]) for t in range(4)]
    Wo = [wo_cat(Wl0, Wr0)] + [wo_cat(Wl[t], Wr[t]) for t in range(4)]
    bsf = [(bl0[0] + bl0[2] + bl0[3])[None, :]] + \
          [(bl[t, 0] + bl[t, 2] + bl[t, 3])[None, :] for t in range(4)]
    bsi = [bl0[1][None, :]] + [bl[t, 1][None, :] for t in range(4)]
    bso = [bl0[4][None, :]] + [bl[t, 4][None, :] for t in range(4)]
    lnw2 = ln_w[None, :]
    lnb2 = ln_b[None, :]

    Pff, Pfi, Pfo, Rf = _prep0(x_function, mask2, Wf[0], 4)
    Pif, Ri = _prep0(x_input, None, Wi[0], 2)
    Pof, Ro = _prep0(x_output, None, Wo[0], 2)

    sd_full = None
    layer_full, rels_full = _build_sc_layer(sd_full, last=False)
    layer_last, rels_last = _build_sc_layer(sd_full, last=True)

    xf_fin = None
    for l in range(5):
        last = l == 4
        if not last:
            o_if, o_ff, o_of, o_fi, o_fo = layer_full(
                Pif, Pff, Pof, Pfi, Pfo,
                sif, sff, sof, sfi, sfo,
                dif, dff, dof, dfi, dfo, zeros32)
            a_if = o_if.reshape(NCORE, ACC_F, H)
            a_ff = o_ff.reshape(NCORE, ACC_F, H)
            a_of = o_of.reshape(NCORE, ACC_F, H)
            a_fi = o_fi.reshape(NCORE, ACC_IO, H)
            a_fo = o_fo.reshape(NCORE, ACC_IO, H)
            if l < 3:
                Pff, Pfi, Pfo, Rf = _combine_f(
                    (a_if, a_ff, a_of), (c_if, c_ff, c_of), Rf, mask2,
                    bsf[l], lnw2, lnb2, Wf[l + 1], 4)
                Pif, Ri = _combine_io(a_fi, c_fi, Ri, bsi[l], lnw2, lnb2,
                                      Wi[l + 1], 2)
                Pof, Ro = _combine_io(a_fo, c_fo, Ro, bso[l], lnw2, lnb2,
                                      Wo[l + 1], 2)
            else:
                w4 = jnp.concatenate([Wl[3][2], Wr[3][0] + Wr[3][2] + Wr[3][3]],
                                     axis=1)
                Pff, Rf = _combine_f(
                    (a_if, a_ff, a_of), (c_if, c_ff, c_of), Rf, mask2,
                    bsf[3], lnw2, lnb2, w4, 2)
                (Pif,) = _combine_io(a_fi, c_fi, Ri, bsi[3], lnw2, lnb2,
                                     Wl[3][0], 1)
                (Pof,) = _combine_io(a_fo, c_fo, Ro, bso[3], lnw2, lnb2,
                                     Wl[3][3], 1)
        else:
            o_if, o_ff, o_of = layer_last(
                Pif, Pff, Pof, sif, sff, sof, dif, dff, dof, zeros32)
            a_if = o_if.reshape(NCORE, ACC_F, H)
            a_ff = o_ff.reshape(NCORE, ACC_F, H)
            a_of = o_of.reshape(NCORE, ACC_F, H)
            (xf_fin,) = _combine_f(
                (a_if, a_ff, a_of), (c_if, c_ff, c_of), Rf, mask2,
                bsf[4], lnw2, lnb2, None, 1)

    return _pool(xf_fin, mask2, batch2, att_w,
                 lin_w, lin_b[None, :].astype(f32))


# speedup vs baseline: 7.6473x; 1.0606x over previous
import functools
import math

import jax
import jax.numpy as jnp
from jax import lax
from jax.experimental import pallas as pl
from jax.experimental.pallas import tpu as pltpu
from jax.experimental.pallas import tpu_sc as plsc

N_IN = 10000
N_FN = 50000
N_OUT = 10000
D = 128
H = 32
NH = 10
NG = 32

NCORE = 2
NSUB = 16
NW = NCORE * NSUB
SUB = 128
CHUNK = 128
KSUB = CHUNK // SUB
PADROWS = 128
ACC_F = 51200
ACC_IO = 10240
CNTW = 16
BLK = 2000

_INV_SQRT2 = 1.0 / math.sqrt(2.0)


def _gelu(x):
    return 0.5 * x * (1.0 + lax.erf(x * _INV_SQRT2))


IB = 10


def _round_edges(e):
    q = IB * NW * CHUNK
    return ((e + q - 1) // q) * q


def _sc_mesh():
    return plsc.VectorSubcoreMesh(core_axis_name="c", subcore_axis_name="s")


def _zero_acc(acc, zv, si, nrows):
    zper = nrows // NSUB // SUB

    def zb(i, c):
        pltpu.sync_copy(zv, acc.at[pl.ds((si * zper + i) * SUB, SUB)])
        return c

    lax.fori_loop(0, zper, zb, 0)


def _flush_acc(acc, out, ci, si, acc_rows):
    share = acc_rows // NSUB
    pltpu.sync_copy(acc.at[pl.ds(si * share, share)],
                    out.at[pl.ds(ci * acc_rows + si * share, share)])


def _agg_rel(P, s2d, d2d, out, n_dst, acc_rows, ci, si,
             acc, sidx, didx, rows, zv, sem0, sem1, ssem0, ssem1):
    _zero_acc(acc, zv, si, acc_rows)
    plsc.subcore_barrier()

    ew = (s2d.shape[0] * SUB) // NW
    nsb = ew // (CHUNK * IB)
    wrow = (ci * NSUB + si) * (ew // SUB)
    gsems = (sem0, sem1)
    ssems = (ssem0, ssem1)

    def load_idx(b, q):
        r0 = wrow + b * IB
        pltpu.sync_copy(s2d.at[pl.ds(r0, IB)], sidx.at[q])
        pltpu.sync_copy(d2d.at[pl.ds(r0, IB)], didx.at[q])

    def fire(q, i):
        p = i % 2
        pltpu.async_copy(P.at[sidx.at[q, i]], rows.at[p], gsems[p])

    def drain_fire_scatter(q, i):
        p = i % 2
        pltpu.make_async_copy(P.at[sidx.at[q, i]], rows.at[p],
                              gsems[p]).wait()
        pltpu.async_copy(rows.at[p], acc.at[didx.at[q, i]], ssems[p],
                         add=True)

    def wait_scatter(p):
        pltpu.make_async_copy(rows.at[p], acc.at[didx.at[0, 0]],
                              ssems[p]).wait()

    load_idx(0, 0)
    fire(0, 0)
    for i in range(1, IB):
        if i >= 2:
            wait_scatter(i % 2)
        fire(0, i)
        drain_fire_scatter(0, i - 1)

    def body(b, c):
        q = lax.rem(b, 2)
        load_idx(b, q)
        for i in range(IB):
            wait_scatter(i % 2)
            fire(q, i)
            if i == 0:
                drain_fire_scatter(1 - q, IB - 1)
            else:
                drain_fire_scatter(q, i - 1)
        return c

    lax.fori_loop(1, nsb, body, 0)
    drain_fire_scatter((nsb - 1) % 2, IB - 1)
    wait_scatter(0)
    wait_scatter(1)
    plsc.subcore_barrier()
    _flush_acc(acc, out, ci, si, acc_rows)
    plsc.subcore_barrier()


def _build_sc_layer(shapes_sd, last):
    rels = ["if", "ff", "of"] + ([] if last else ["fi", "fo"])
    ndst = {"if": N_FN, "ff": N_FN, "of": N_FN, "fi": N_IN, "fo": N_OUT}
    accr = {"if": ACC_F, "ff": ACC_F, "of": ACC_F, "fi": ACC_IO, "fo": ACC_IO}

    out_type = [jax.ShapeDtypeStruct((NCORE * accr[r], H), jnp.float32)
                for r in rels]

    @functools.partial(
        pl.kernel,
        out_type=out_type,
        mesh=_sc_mesh(),
        compiler_params=pltpu.CompilerParams(use_tc_tiling_on_sc=False),
        scratch_types=[
            pltpu.VMEM_SHARED((ACC_F, H), jnp.float32),
            pltpu.VMEM((2, IB, SUB), jnp.int32),
            pltpu.VMEM((2, IB, SUB), jnp.int32),
            pltpu.VMEM((2, SUB, H), jnp.float32),
            pltpu.VMEM((SUB, H), jnp.float32),
            pltpu.SemaphoreType.DMA,
            pltpu.SemaphoreType.DMA,
            pltpu.SemaphoreType.DMA,
            pltpu.SemaphoreType.DMA,
        ],
    )
    def k(*args):
        nr = len(rels)
        Ps = args[0:nr]
        s2ds = args[nr:2 * nr]
        d2ds = args[2 * nr:3 * nr]
        zeros_hbm = args[3 * nr]
        outs = args[3 * nr + 1:3 * nr + 1 + nr]
        (acc, sidx, didx, rows, zv,
         sem0, sem1, ssem0, ssem1) = args[3 * nr + 1 + nr:]
        ci = lax.axis_index("c")
        si = lax.axis_index("s")
        pltpu.sync_copy(zeros_hbm, zv)
        for t, r in enumerate(rels):
            _agg_rel(Ps[t], s2ds[t], d2ds[t], outs[t], ndst[r], accr[r],
                     ci, si, acc, sidx, didx, rows, zv,
                     sem0, sem1, ssem0, ssem1)

    return k, rels


def _build_sc_counts():
    rels = ["if", "ff", "of", "fi", "fo"]
    ndst = {"if": N_FN, "ff": N_FN, "of": N_FN, "fi": N_IN, "fo": N_OUT}
    accr = {"if": ACC_F, "ff": ACC_F, "of": ACC_F, "fi": ACC_IO, "fo": ACC_IO}

    out_type = [jax.ShapeDtypeStruct((NCORE * accr[r], CNTW), jnp.float32)
                for r in rels]

    @functools.partial(
        pl.kernel,
        out_type=out_type,
        mesh=_sc_mesh(),
        compiler_params=pltpu.CompilerParams(use_tc_tiling_on_sc=False),
        scratch_types=[
            pltpu.VMEM_SHARED((ACC_F, CNTW), jnp.float32),
            pltpu.VMEM((8, SUB), jnp.int32),
            pltpu.VMEM((SUB, CNTW), jnp.float32),
            pltpu.VMEM((SUB, CNTW), jnp.float32),
        ],
    )
    def k(*args):
        d2ds = args[0:5]
        zeros_hbm = args[5]
        ones_hbm = args[6]
        outs = args[7:12]
        acc, didx, zv, ones_v = args[12:]
        ci = lax.axis_index("c")
        si = lax.axis_index("s")
        pltpu.sync_copy(zeros_hbm, zv)
        pltpu.sync_copy(ones_hbm, ones_v)
        for t, r in enumerate(rels):
            _zero_acc(acc, zv, si, accr[r])
            plsc.subcore_barrier()
            d2d = d2ds[t]
            ew = (d2d.shape[0] * SUB) // NW
            nchunks = (ew // SUB) // 8
            wrow = (ci * NSUB + si) * (ew // SUB)

            def cb(kk, c, d2d=d2d, wrow=wrow):
                r0 = wrow + kk * 8
                pltpu.sync_copy(d2d.at[pl.ds(r0, 8)], didx)
                for j in range(8):
                    pltpu.sync_copy(ones_v, acc.at[didx.at[j]], add=True)
                return c

            lax.fori_loop(0, nchunks, cb, 0)
            plsc.subcore_barrier()
            _flush_acc(acc, outs[t], ci, si, accr[r])
            plsc.subcore_barrier()

    return k


def _dot(a, b):
    return jnp.dot(a, b, preferred_element_type=jnp.float32)


def _prep0(x, mask2, w, nout):
    n = x.shape[0]
    grid = n // BLK
    have_mask = mask2 is not None

    def body(*refs):
        if have_mask:
            x_ref, m_ref, w_ref = refs[:3]
            outs = refs[3:]
            xv = x_ref[...] * m_ref[...]
        else:
            x_ref, w_ref = refs[:2]
            outs = refs[2:]
            xv = x_ref[...]
        y = _dot(xv, w_ref[...])
        for t in range(nout):
            outs[t][...] = y[:, H * t:H * (t + 1)]

    in_specs = [pl.BlockSpec((BLK, D), lambda i: (i, 0))]
    args = [x]
    if have_mask:
        in_specs.append(pl.BlockSpec((BLK, 1), lambda i: (i, 0)))
        args.append(mask2)
    in_specs.append(pl.BlockSpec((D, H * nout), lambda i: (0, 0)))
    args.append(w)

    return pl.pallas_call(
        body,
        grid=(grid,),
        in_specs=in_specs,
        out_specs=[pl.BlockSpec((BLK, H), lambda i: (i, 0))] * nout,
        out_shape=[jax.ShapeDtypeStruct((n, H), jnp.float32)] * nout,
    )(*args)


def _mean_blk(a_ref, c_ref):
    s = a_ref[0] + a_ref[1]
    cnt = c_ref[0][:, 0:1] + c_ref[1][:, 0:1]
    return s / jnp.maximum(cnt, 1.0)


def _ln_gelu(nf, lw_ref, lb_ref):
    g = _gelu(nf)
    mu = jnp.mean(g, axis=-1, keepdims=True)
    var = jnp.mean((g - mu) ** 2, axis=-1, keepdims=True)
    return (g - mu) / jnp.sqrt(var + 1e-5) * lw_ref[...] + lb_ref[...]


def _combine_f(aggs, cnts, r, mask2, bsum, lnw, lnb, w, nout):
    n = r.shape[0]
    grid = n // BLK

    def body(a0, c0, a1, c1, a2, c2, r_ref, m_ref, b_ref, lw, lb, *rest):
        if w is not None:
            w_ref = rest[0]
            outs = rest[1:]
        else:
            outs = rest
        nf = (_mean_blk(a0, c0) + _mean_blk(a1, c1) + _mean_blk(a2, c2)
              + r_ref[...] + b_ref[...])
        nf = nf * m_ref[...]
        x = _ln_gelu(nf, lw, lb)
        if w is not None:
            y = _dot(x, w_ref[...])
            for t in range(nout):
                outs[t][...] = y[:, H * t:H * (t + 1)]
        else:
            outs[0][...] = x

    a_spec = pl.BlockSpec((NCORE, BLK, H), lambda i: (0, i, 0))
    c_spec = pl.BlockSpec((NCORE, BLK, CNTW), lambda i: (0, i, 0))
    v_spec = pl.BlockSpec((BLK, H), lambda i: (i, 0))
    s_spec = pl.BlockSpec((1, H), lambda i: (0, 0))
    in_specs = [a_spec, c_spec, a_spec, c_spec, a_spec, c_spec,
                v_spec, pl.BlockSpec((BLK, 1), lambda i: (i, 0)),
                s_spec, s_spec, s_spec]
    args = [aggs[0], cnts[0], aggs[1], cnts[1], aggs[2], cnts[2],
            r, mask2, bsum, lnw, lnb]
    if w is not None:
        in_specs.append(pl.BlockSpec((H, H * nout), lambda i: (0, 0)))
        args.append(w)

    return pl.pallas_call(
        body,
        grid=(grid,),
        in_specs=in_specs,
        out_specs=[v_spec] * nout,
        out_shape=[jax.ShapeDtypeStruct((n, H), jnp.float32)] * nout,
    )(*args)


def _combine_io(agg, cnt, r, bsum, lnw, lnb, w, nout):
    n = r.shape[0]
    grid = n // BLK

    def body(a0, c0, r_ref, b_ref, lw, lb, w_ref, *outs):
        nf = _mean_blk(a0, c0) + r_ref[...] + b_ref[...]
        x = _ln_gelu(nf, lw, lb)
        y = _dot(x, w_ref[...])
        for t in range(nout):
            outs[t][...] = y[:, H * t:H * (t + 1)]

    v_spec = pl.BlockSpec((BLK, H), lambda i: (i, 0))
    s_spec = pl.BlockSpec((1, H), lambda i: (0, 0))
    in_specs = [pl.BlockSpec((NCORE, BLK, H), lambda i: (0, i, 0)),
                pl.BlockSpec((NCORE, BLK, CNTW), lambda i: (0, i, 0)),
                v_spec, s_spec, s_spec, s_spec,
                pl.BlockSpec((H, H * nout), lambda i: (0, 0))]

    return pl.pallas_call(
        body,
        grid=(grid,),
        in_specs=in_specs,
        out_specs=[v_spec] * nout,
        out_shape=[jax.ShapeDtypeStruct((n, H), jnp.float32)] * nout,
    )(agg, cnt, r, bsum, lnw, lnb, w)


def _pool(xf, mask2, batch2, att_w, lin_w, lin_b):
    nblk = N_FN // BLK

    def body(xf_ref, m_ref, b_ref, aw_ref, lw_ref, lb_ref, o_ref,
             smax_s, den_s, num_s):
        p = pl.program_id(0)
        i = pl.program_id(1)
        xfm = xf_ref[...] * m_ref[...]
        gid = lax.broadcasted_iota(jnp.int32, (BLK, NG), 1)
        oneh = (b_ref[...] == gid).astype(jnp.float32)
        scores = _dot(xfm, aw_ref[...])
        neg = jnp.float32(-jnp.inf)

        @pl.when(p == 0)
        def _phase0():
            rows = []
            for g in range(NG):
                mg = jnp.where(oneh[:, g:g + 1] > 0.0, scores, neg)
                rows.append(jnp.max(mg, axis=0, keepdims=True))
            bm = jnp.concatenate(rows, axis=0)

            @pl.when(i == 0)
            def _():
                smax_s[...] = bm

            @pl.when(i > 0)
            def _():
                smax_s[...] = jnp.maximum(smax_s[...], bm)

        @pl.when(p == 1)
        def _phase1():
            smax = smax_s[...]
            smax = jnp.where(jnp.isfinite(smax), smax, 0.0)
            shift = _dot(oneh, smax)
            ex = jnp.exp(scores - shift)
            den = lax.dot_general(oneh, ex, (((0,), (0,)), ((), ())),
                                  preferred_element_type=jnp.float32)
            nums = []
            for h in range(NH):
                wh = oneh * ex[:, h:h + 1]
                nums.append(lax.dot_general(wh, xfm, (((0,), (0,)), ((), ())),
                                            preferred_element_type=jnp.float32))
            num = jnp.concatenate(nums, axis=0)

            @pl.when(i == 0)
            def _():
                den_s[...] = den
                num_s[...] = num

            @pl.when(i > 0)
            def _():
                den_s[...] = den_s[...] + den
                num_s[...] = num_s[...] + num

        @pl.when((p == 1) & (i == nblk - 1))
        def _epilogue():
            den = jnp.maximum(den_s[...], 1e-9)
            acc = jnp.zeros((NG, 1), jnp.float32)
            for h in range(NH):
                ph = num_s[h * NG:(h + 1) * NG, :] / den[:, h:h + 1]
                acc = acc + _dot(_gelu(ph), lw_ref[h * H:(h + 1) * H, :])
            o_ref[...] = acc + lb_ref[...]

    return pl.pallas_call(
        body,
        grid=(2, nblk),
        in_specs=[pl.BlockSpec((BLK, H), lambda p, i: (i, 0)),
                  pl.BlockSpec((BLK, 1), lambda p, i: (i, 0)),
                  pl.BlockSpec((BLK, 1), lambda p, i: (i, 0)),
                  pl.BlockSpec(att_w.shape, lambda p, i: (0, 0)),
                  pl.BlockSpec(lin_w.shape, lambda p, i: (0, 0)),
                  pl.BlockSpec(lin_b.shape, lambda p, i: (0, 0))],
        out_specs=pl.BlockSpec((NG, 1), lambda p, i: (0, 0)),
        out_shape=jax.ShapeDtypeStruct((NG, 1), jnp.float32),
        scratch_shapes=[pltpu.VMEM((NG, NH), jnp.float32),
                        pltpu.VMEM((NG, NH), jnp.float32),
                        pltpu.VMEM((NH * NG, H), jnp.float32)],
    )(xf, mask2, batch2, att_w, lin_w, lin_b)


def kernel(x_input, x_function, x_output, edge_index_if, edge_index_fi,
           edge_index_ff, edge_index_of, edge_index_fo, batch, mask,
           Wl0, bl0, Wr0, Wl, bl, Wr, ln_w, ln_b, att_w, lin_w, lin_b):
    f32 = jnp.float32
    mask2 = mask[:, None].astype(f32)
    batch2 = batch[:, None].astype(jnp.int32)
    zeros32 = jnp.zeros((SUB, H), f32)
    zeros16 = jnp.zeros((SUB, CNTW), f32)
    ones16 = jnp.ones((SUB, CNTW), f32)

    def prep_edges(ei, n_src, n_dst):
        e = ei.shape[1]
        ep = _round_edges(e)
        pad = ep - e
        ar = jnp.arange(pad, dtype=jnp.int32)
        s = jnp.concatenate([ei[0].astype(jnp.int32), ar % n_src])
        dd = jnp.concatenate([ei[1].astype(jnp.int32),
                              n_dst + (ar % PADROWS)])
        return s.reshape(ep // SUB, SUB), dd.reshape(ep // SUB, SUB)

    sif, dif = prep_edges(edge_index_if, N_IN, N_FN)
    sff, dff = prep_edges(edge_index_ff, N_FN, N_FN)
    sof, dof = prep_edges(edge_index_of, N_OUT, N_FN)
    sfi, dfi = prep_edges(edge_index_fi, N_FN, N_IN)
    sfo, dfo = prep_edges(edge_index_fo, N_FN, N_OUT)

    counts_k = _build_sc_counts()
    cr = counts_k(dif, dff, dof, dfi, dfo, zeros16, ones16)
    c_if, c_ff, c_of, c_fi, c_fo = [
        c.reshape(NCORE, n, CNTW)
        for c, n in zip(cr, (ACC_F, ACC_F, ACC_F, ACC_IO, ACC_IO))]

    def wf_cat(WL, WR):
        return jnp.concatenate([WL[2], WL[1], WL[4], WR[0] + WR[2] + WR[3]],
                               axis=1)

    def wi_cat(WL, WR):
        return jnp.concatenate([WL[0], WR[1]], axis=1)

    def wo_cat(WL, WR):
        return jnp.concatenate([WL[3], WR[4]], axis=1)

    Wf = [wf_cat(Wl0, Wr0)] + [wf_cat(Wl[t], Wr[t]) for t in range(4)]
    Wi = [wi_cat(Wl0, Wr0)] + [wi_cat(Wl[t], Wr[t]) for t in range(4)]
    Wo = [wo_cat(Wl0, Wr0)] + [wo_cat(Wl[t], Wr[t]) for t in range(4)]
    bsf = [(bl0[0] + bl0[2] + bl0[3])[None, :]] + \
          [(bl[t, 0] + bl[t, 2] + bl[t, 3])[None, :] for t in range(4)]
    bsi = [bl0[1][None, :]] + [bl[t, 1][None, :] for t in range(4)]
    bso = [bl0[4][None, :]] + [bl[t, 4][None, :] for t in range(4)]
    lnw2 = ln_w[None, :]
    lnb2 = ln_b[None, :]

    Pff, Pfi, Pfo, Rf = _prep0(x_function, mask2, Wf[0], 4)
    Pif, Ri = _prep0(x_input, None, Wi[0], 2)
    Pof, Ro = _prep0(x_output, None, Wo[0], 2)

    sd_full = None
    layer_full, rels_full = _build_sc_layer(sd_full, last=False)
    layer_last, rels_last = _build_sc_layer(sd_full, last=True)

    xf_fin = None
    for l in range(5):
        last = l == 4
        if not last:
            o_if, o_ff, o_of, o_fi, o_fo = layer_full(
                Pif, Pff, Pof, Pfi, Pfo,
                sif, sff, sof, sfi, sfo,
                dif, dff, dof, dfi, dfo, zeros32)
            a_if = o_if.reshape(NCORE, ACC_F, H)
            a_ff = o_ff.reshape(NCORE, ACC_F, H)
            a_of = o_of.reshape(NCORE, ACC_F, H)
            a_fi = o_fi.reshape(NCORE, ACC_IO, H)
            a_fo = o_fo.reshape(NCORE, ACC_IO, H)
            if l < 3:
                Pff, Pfi, Pfo, Rf = _combine_f(
                    (a_if, a_ff, a_of), (c_if, c_ff, c_of), Rf, mask2,
                    bsf[l], lnw2, lnb2, Wf[l + 1], 4)
                Pif, Ri = _combine_io(a_fi, c_fi, Ri, bsi[l], lnw2, lnb2,
                                      Wi[l + 1], 2)
                Pof, Ro = _combine_io(a_fo, c_fo, Ro, bso[l], lnw2, lnb2,
                                      Wo[l + 1], 2)
            else:
                w4 = jnp.concatenate([Wl[3][2], Wr[3][0] + Wr[3][2] + Wr[3][3]],
                                     axis=1)
                Pff, Rf = _combine_f(
                    (a_if, a_ff, a_of), (c_if, c_ff, c_of), Rf, mask2,
                    bsf[3], lnw2, lnb2, w4, 2)
                (Pif,) = _combine_io(a_fi, c_fi, Ri, bsi[3], lnw2, lnb2,
                                     Wl[3][0], 1)
                (Pof,) = _combine_io(a_fo, c_fo, Ro, bso[3], lnw2, lnb2,
                                     Wl[3][3], 1)
        else:
            o_if, o_ff, o_of = layer_last(
                Pif, Pff, Pof, sif, sff, sof, dif, dff, dof, zeros32)
            a_if = o_if.reshape(NCORE, ACC_F, H)
            a_ff = o_ff.reshape(NCORE, ACC_F, H)
            a_of = o_of.reshape(NCORE, ACC_F, H)
            (xf_fin,) = _combine_f(
                (a_if, a_ff, a_of), (c_if, c_ff, c_of), Rf, mask2,
                bsf[4], lnw2, lnb2, None, 1)

    return _pool(xf_fin, mask2, batch2, att_w,
                 lin_w, lin_b[None, :].astype(f32))
